# baseline (speedup 1.0000x reference)
import functools

import jax
import jax.numpy as jnp
from jax import lax
from jax.experimental import pallas as pl
from jax.experimental.pallas import tpu as pltpu
from jax.experimental.pallas import tpu_sc as plsc

N = 100000
D = 512
K = N // 2
NWS = 16
NLW = 32
CH = 6400
LCH = CH // 2
NP = NWS * CH
NVS = LCH // 16
KP = 50048
KCH = KP // NWS

_KBLK = 2048


def _keys_body(x_ref, w_ref, b_ref, o_ref):
    s = jnp.sum(x_ref[...] * w_ref[...], axis=1) + b_ref[0]
    bits = lax.bitcast_convert_type(s, jnp.int32)
    key = jnp.where(bits >= 0, bits ^ 0x7FFFFFFF, bits)
    o_ref[...] = lax.bitcast_convert_type(key, jnp.uint32)


def _keys_pallas(x, W, b):
    return pl.pallas_call(
        _keys_body,
        grid=(pl.cdiv(N, _KBLK),),
        in_specs=[
            pl.BlockSpec((_KBLK, D), lambda i: (i, 0)),
            pl.BlockSpec((1, D), lambda i: (0, 0)),
            pl.BlockSpec(memory_space=pltpu.SMEM),
        ],
        out_specs=pl.BlockSpec((_KBLK,), lambda i: (i,)),
        out_shape=jax.ShapeDtypeStruct((N,), jnp.uint32),
    )(x, W.reshape(1, D), b)


def _mesh():
    return plsc.VectorSubcoreMesh(core_axis_name="c", subcore_axis_name="s")


_SC_PARAMS = pltpu.CompilerParams(needs_layout_passes=False)


def _as_i32(v):
    return v if v.dtype == jnp.int32 else plsc.bitcast(v, jnp.int32)


@functools.partial(
    pl.kernel,
    out_type=jax.ShapeDtypeStruct((KP,), jnp.int32),
    mesh=_mesh(),
    compiler_params=_SC_PARAMS,
    scratch_types=[
        pltpu.VMEM((CH,), jnp.uint32),
        pltpu.VMEM((CH,), jnp.int32),
        pltpu.VMEM((512,), jnp.int32),
        pltpu.VMEM((NLW * 256,), jnp.int32),
        pltpu.VMEM((256,), jnp.int32),
        pltpu.VMEM((256,), jnp.int32),
        pltpu.VMEM_SHARED((NP,), jnp.uint32),
        pltpu.VMEM_SHARED((NP,), jnp.uint32),
        pltpu.VMEM_SHARED((NP,), jnp.int32),
        pltpu.VMEM_SHARED((NP,), jnp.int32),
        pltpu.VMEM_SHARED((NLW * 256,), jnp.int32),
        pltpu.SemaphoreType.DMA,
        pltpu.SemaphoreType.DMA,
    ],
)
def _radix_sort(keys_hbm, idx_hbm, out_hbm, kc, ic, hist_v, tloc,
                cnt_a, cnt_b, kb0, kb1, ib0, ib1, htab, sem_k, sem_i):
    cid = lax.axis_index("c")
    w = lax.axis_index("s")

    @pl.when(cid == 0)
    def _():
        base = w * CH
        zero16 = jnp.zeros((16,), jnp.int32)
        plan = [
            (0, keys_hbm, idx_hbm, kb0, ib0),
            (8, kb0, ib0, kb1, ib1),
            (16, kb1, ib1, kb0, ib0),
            (24, kb0, ib0, None, ib1),
        ]
        for shift, srck, srci, dstk, dsti in plan:
            pltpu.sync_copy(srck.at[pl.ds(base, CH)], kc)
            pltpu.sync_copy(srci.at[pl.ds(base, CH)], ic)

            def digit(kv):
                return _as_i32((kv >> jnp.uint32(shift)) & jnp.uint32(0xFF))

            for g in range(32):
                hist_v[pl.ds(g * 16, 16)] = zero16

            @pl.loop(0, NVS)
            def _h(i):
                for half in (0, 1):
                    da = digit(kc[pl.ds(half * LCH + i * 16, 16)])
                    hv = hist_v.at[pl.ds(half * 256, 256)]
                    cnt, last = plsc.scan_count(da)
                    bs = plsc.load_gather(hv, [da])
                    plsc.store_scatter(hv, [da], bs + _as_i32(cnt), mask=last)

            pltpu.sync_copy(hist_v, htab.at[pl.ds(w * 512, 512)])
            plsc.subcore_barrier()

            pltpu.sync_copy(htab, tloc)
            carry = jnp.int32(0)
            for g in range(16):
                colsum = jnp.zeros((16,), jnp.int32)
                presum_a = jnp.zeros((16,), jnp.int32)
                presum_b = jnp.zeros((16,), jnp.int32)
                for wp in range(NLW):
                    v = tloc[pl.ds(wp * 256 + g * 16, 16)]
                    colsum = colsum + v
                    ma = jnp.broadcast_to(
                        (2 * w > wp).astype(jnp.int32), (16,))
                    mb = jnp.broadcast_to(
                        (2 * w + 1 > wp).astype(jnp.int32), (16,))
                    presum_a = presum_a + v * ma
                    presum_b = presum_b + v * mb
                incl = plsc.cumsum(colsum)
                excl = (incl - colsum) + carry
                cnt_a[pl.ds(g * 16, 16)] = excl + presum_a
                cnt_b[pl.ds(g * 16, 16)] = excl + presum_b
                carry = carry + jnp.sum(colsum)

            @pl.loop(0, NVS)
            def _r(i):
                for half, cnt_v in ((0, cnt_a), (1, cnt_b)):
                    sl = pl.ds(half * LCH + i * 16, 16)
                    da = digit(kc[sl])
                    cnt, last = plsc.scan_count(da)
                    cnti = _as_i32(cnt)
                    bs = plsc.load_gather(cnt_v, [da])
                    pos = bs + cnti - 1
                    plsc.store_scatter(cnt_v, [da], bs + cnti, mask=last)
                    if dstk is not None:
                        pltpu.async_copy(kc.at[sl], dstk.at[pos], sem_k)
                    pltpu.async_copy(ic.at[sl], dsti.at[pos], sem_i)

            if dstk is not None:
                pltpu.make_async_copy(
                    keys_hbm.at[pl.ds(0, CH)], kc, sem_k).wait()
            pltpu.make_async_copy(idx_hbm.at[pl.ds(0, CH)], ic, sem_i).wait()
            plsc.subcore_barrier()

        pltpu.sync_copy(ib1.at[pl.ds(w * KCH, KCH)], ic.at[pl.ds(0, KCH)])
        pltpu.sync_copy(ic.at[pl.ds(0, KCH)], out_hbm.at[pl.ds(w * KCH, KCH)])


_R = 80
_NWG = 32
_FULL_T = K // (_NWG * _R)
_REM_W = (K - _FULL_T * _NWG * _R) // _R


@functools.partial(
    pl.kernel,
    out_type=jax.ShapeDtypeStruct((K, D), jnp.float32),
    mesh=_mesh(),
    compiler_params=_SC_PARAMS,
    scratch_types=[
        pltpu.VMEM((_R,), jnp.int32),
        pltpu.VMEM((_R,), jnp.int32),
        pltpu.VMEM((_R, D), jnp.float32),
        pltpu.VMEM((_R, D), jnp.float32),
        pltpu.SemaphoreType.DMA,
        pltpu.SemaphoreType.DMA,
        pltpu.SemaphoreType.DMA,
        pltpu.SemaphoreType.DMA,
    ],
)
def _gather_rows(x_hbm, sidx_hbm, out_hbm, idx_v0, idx_v1, rows_v0, rows_v1,
                 g0, g1, s0, s1):
    w = lax.axis_index("c") * 16 + lax.axis_index("s")
    idx_v = (idx_v0, idx_v1)
    rows_v = (rows_v0, rows_v1)
    gsem = (g0, g1)
    ssem = (s0, s1)

    def off(t):
        return (t * _NWG + w) * _R

    def start_gather(t, buf):
        pltpu.sync_copy(sidx_hbm.at[pl.ds(off(t), _R)], idx_v[buf])
        return pltpu.async_copy(x_hbm.at[idx_v[buf]], rows_v[buf], gsem[buf])

    def start_store(t, buf):
        return pltpu.async_copy(rows_v[buf],
                                out_hbm.at[pl.ds(off(t), _R)], ssem[buf])

    gathers = [start_gather(0, 0), start_gather(1, 1)]
    stores = [None, None]
    for t in range(_FULL_T):
        buf = t % 2
        gathers[buf].wait()
        stores[buf] = start_store(t, buf)
        if t + 2 < _FULL_T:
            stores[buf].wait()
            gathers[buf] = start_gather(t + 2, buf)
    for t in (_FULL_T - 2, _FULL_T - 1):
        stores[t % 2].wait()

    @pl.when(w < _REM_W)
    def _():
        g = start_gather(_FULL_T, 0)
        g.wait()
        start_store(_FULL_T, 0).wait()


_PALLAS_SCORES = False


def kernel(x, W, b):
    if _PALLAS_SCORES:
        keys = _keys_pallas(x, W, b)
    else:
        scores = jnp.squeeze(x @ W + b)
        bits = lax.bitcast_convert_type(scores, jnp.int32)
        key_i = jnp.where(bits >= 0, bits ^ 0x7FFFFFFF, bits)
        keys = lax.bitcast_convert_type(key_i, jnp.uint32)

    keys_p = jnp.concatenate(
        [keys, jnp.full((NP - N,), 0xFFFFFFFF, jnp.uint32)])
    idx_p = jnp.arange(NP, dtype=jnp.int32)

    sidx = _radix_sort(keys_p, idx_p)
    rows = _gather_rows(x, sidx)
    return rows, sidx[:K]
